# Initial kernel scaffold; baseline (speedup 1.0000x reference)
import jax
import jax.numpy as jnp
from jax.experimental import pallas as pl


def kernel(hidden_states, gate_weight, w_gate, w_up, w_down, sw_gate, sw_up, sw_down):
    raise NotImplementedError("write your pallas kernel here")


# speedup vs baseline: 1.7930x; 1.7930x over previous
import jax
import jax.numpy as jnp
from jax.experimental import pallas as pl
from jax.experimental.pallas import tpu as pltpu

TT = 256


def _router_body(x_ref, gwt_ref, gates_ref):
    x = x_ref[...]
    lg = jnp.dot(x, gwt_ref[...], preferred_element_type=jnp.float32)
    m = jnp.max(lg, axis=-1, keepdims=True)
    ex = jnp.exp(lg - m)
    s = ex / jnp.sum(ex, axis=-1, keepdims=True)
    E = s.shape[-1]
    neg = jnp.float32(-1.0)
    m1 = s[:, 0:1]
    i1 = jnp.zeros_like(m1, dtype=jnp.int32)
    m2 = jnp.full_like(m1, neg)
    i2 = jnp.full_like(i1, -1)
    for e in range(1, E):
        v = s[:, e : e + 1]
        gt1 = v > m1
        gt2 = v > m2
        m2n = jnp.where(gt1, m1, jnp.where(gt2, v, m2))
        i2n = jnp.where(gt1, i1, jnp.where(gt2, e, i2))
        m1 = jnp.where(gt1, v, m1)
        i1 = jnp.where(gt1, e, i1)
        m2, i2 = m2n, i2n
    for e in range(E):
        ge = jnp.where(i1 == e, m1, 0.0) + jnp.where(i2 == e, m2, 0.0)
        gates_ref[:, e : e + 1] = ge


def _expert_body(x_ref, wg_ref, wu_ref, wd_ref, out_ref):
    x = x_ref[...]
    xg = jnp.dot(x, wg_ref[0], preferred_element_type=jnp.float32)
    xu = jnp.dot(x, wu_ref[0], preferred_element_type=jnp.float32)
    h = (xg * jax.nn.sigmoid(xg) * xu).astype(jnp.bfloat16)
    out_ref[0] = jnp.dot(h, wd_ref[0], preferred_element_type=jnp.float32)


def _combine_body(x_ref, part_ref, gates_ref, swg_ref, swu_ref, swd_ref, y_ref):
    x = x_ref[...]
    xg = jnp.dot(x, swg_ref[...], preferred_element_type=jnp.float32)
    xu = jnp.dot(x, swu_ref[...], preferred_element_type=jnp.float32)
    h = (xg * jax.nn.sigmoid(xg) * xu).astype(jnp.bfloat16)
    acc = jnp.dot(h, swd_ref[...], preferred_element_type=jnp.float32)
    E = gates_ref.shape[-1]
    for e in range(E):
        acc = acc + part_ref[e] * gates_ref[:, e : e + 1]
    y_ref[0] = acc


def kernel(hidden_states, gate_weight, w_gate, w_up, w_down, sw_gate, sw_up, sw_down):
    Bsz, S, D = hidden_states.shape
    E, _, F = w_gate.shape
    FS = sw_gate.shape[1]
    N = Bsz * S
    nt = N // TT

    x = hidden_states.reshape(N, D)
    x16 = x.astype(jnp.bfloat16)
    gwt = gate_weight.T
    wg16 = w_gate.astype(jnp.bfloat16)
    wu16 = w_up.astype(jnp.bfloat16)
    wd16 = w_down.astype(jnp.bfloat16)
    swg16 = sw_gate.astype(jnp.bfloat16)
    swu16 = sw_up.astype(jnp.bfloat16)
    swd16 = sw_down.astype(jnp.bfloat16)

    gates = pl.pallas_call(
        _router_body,
        grid=(nt,),
        in_specs=[
            pl.BlockSpec((TT, D), lambda t: (t, 0)),
            pl.BlockSpec((D, E), lambda t: (0, 0)),
        ],
        out_specs=pl.BlockSpec((TT, E), lambda t: (t, 0)),
        out_shape=jax.ShapeDtypeStruct((N, E), jnp.float32),
    )(x, gwt)

    partial = pl.pallas_call(
        _expert_body,
        grid=(E, nt),
        in_specs=[
            pl.BlockSpec((TT, D), lambda e, t: (t, 0)),
            pl.BlockSpec((1, D, F), lambda e, t: (e, 0, 0)),
            pl.BlockSpec((1, D, F), lambda e, t: (e, 0, 0)),
            pl.BlockSpec((1, F, D), lambda e, t: (e, 0, 0)),
        ],
        out_specs=pl.BlockSpec((1, TT, D), lambda e, t: (e, t, 0)),
        out_shape=jax.ShapeDtypeStruct((E, N, D), jnp.float32),
    )(x16, wg16, wu16, wd16)

    y = pl.pallas_call(
        _combine_body,
        grid=(nt,),
        in_specs=[
            pl.BlockSpec((TT, D), lambda t: (t, 0)),
            pl.BlockSpec((E, TT, D), lambda t: (0, t, 0)),
            pl.BlockSpec((TT, E), lambda t: (t, 0)),
            pl.BlockSpec((D, FS), lambda t: (0, 0)),
            pl.BlockSpec((D, FS), lambda t: (0, 0)),
            pl.BlockSpec((FS, D), lambda t: (0, 0)),
        ],
        out_specs=pl.BlockSpec((1, TT, D), lambda t: (0, t, 0)),
        out_shape=jax.ShapeDtypeStruct((1, N, D), jnp.float32),
    )(x16, partial, gates, swg16, swu16, swd16)

    return y.reshape(Bsz, S, D)
